# baseline (speedup 1.0000x reference)
import functools
import jax
import jax.numpy as jnp
from jax import lax
from jax.experimental import pallas as pl
from jax.experimental.pallas import tpu as pltpu
from jax.experimental.pallas import tpu_sc as plsc

T = 2048
D = 1024
H = 16
KV = 4
HD = 64
E = 8
K = 2
FF = 1024
EPS = 1e-6
THETA = 10000.0

BT = 256
NT = T // BT
HALF = HD // 2


def _rms(x, w, eps=EPS):
    var = jnp.mean(x * x, axis=-1, keepdims=True)
    return x * lax.rsqrt(var + eps) * w


def _pre_attn_kernel(x_ref, wqkv_ref, ln1_ref, qn_ref, kn_ref, pos_ref,
                     q_ref, k_ref, v_ref):
    x = x_ref[...]
    h = _rms(x, ln1_ref[...])
    qkv = jnp.dot(h.astype(jnp.bfloat16), wqkv_ref[...],
                  preferred_element_type=jnp.float32)

    pos = pos_ref[...].astype(jnp.float32)
    i2 = lax.broadcasted_iota(jnp.int32, (1, HALF), 1).astype(jnp.float32)
    inv_freq = jnp.exp(i2 * (-2.0 * jnp.log(THETA) / HD))
    freqs = pos * inv_freq
    cos = jnp.cos(freqs)
    sin = jnp.sin(freqs)

    def rope_norm(t, w):
        t = _rms(t, w)
        t1 = t[:, :HALF]
        t2 = t[:, HALF:]
        return jnp.concatenate([t1 * cos - t2 * sin, t2 * cos + t1 * sin], axis=1)

    qn = qn_ref[...]
    kn = kn_ref[...]
    for hh in range(H):
        q_ref[hh, :, :] = rope_norm(qkv[:, hh * HD:(hh + 1) * HD],
                                    qn).astype(jnp.bfloat16)
    for hh in range(KV):
        base = H * HD + hh * HD
        k_ref[hh, :, :] = rope_norm(qkv[:, base:base + HD],
                                    kn).astype(jnp.bfloat16)
        v_ref[hh, :, :] = qkv[:, H * HD + KV * HD + hh * HD:
                              H * HD + KV * HD + (hh + 1) * HD].astype(jnp.bfloat16)


def _pre_attn(x, w_qkv, ln1_w, q_norm_w, k_norm_w, positions):
    return pl.pallas_call(
        _pre_attn_kernel,
        grid=(NT,),
        in_specs=[
            pl.BlockSpec((BT, D), lambda i: (i, 0)),
            pl.BlockSpec((D, (H + 2 * KV) * HD), lambda i: (0, 0)),
            pl.BlockSpec((1, D), lambda i: (0, 0)),
            pl.BlockSpec((1, HD), lambda i: (0, 0)),
            pl.BlockSpec((1, HD), lambda i: (0, 0)),
            pl.BlockSpec((BT, 1), lambda i: (i, 0)),
        ],
        out_specs=[
            pl.BlockSpec((H, BT, HD), lambda i: (0, i, 0)),
            pl.BlockSpec((KV, BT, HD), lambda i: (0, i, 0)),
            pl.BlockSpec((KV, BT, HD), lambda i: (0, i, 0)),
        ],
        out_shape=[
            jax.ShapeDtypeStruct((H, T, HD), jnp.bfloat16),
            jax.ShapeDtypeStruct((KV, T, HD), jnp.bfloat16),
            jax.ShapeDtypeStruct((KV, T, HD), jnp.bfloat16),
        ],
    )(x, w_qkv.astype(jnp.bfloat16), ln1_w.reshape(1, D), q_norm_w.reshape(1, HD),
      k_norm_w.reshape(1, HD), positions.reshape(T, 1))


BKV = 1024


def _attn_kernel(q_ref, k_ref, v_ref, o_ref):
    iq = pl.program_id(0)
    nb = (iq * BT + BT + BKV - 1) // BKV

    for hh in range(H):
        kvh = hh // (H // KV)
        q = q_ref[hh]

        def body(j, carry):
            m, l, acc = carry
            ks = k_ref[kvh, pl.ds(j * BKV, BKV), :]
            vs = v_ref[kvh, pl.ds(j * BKV, BKV), :]
            s = lax.dot_general(q, ks, (((1,), (1,)), ((), ())),
                                preferred_element_type=jnp.float32)
            s = s * (HD ** -0.5)
            row = iq * BT + lax.broadcasted_iota(jnp.int32, (BT, BKV), 0)
            col = j * BKV + lax.broadcasted_iota(jnp.int32, (BT, BKV), 1)
            s = jnp.where(row >= col, s, jnp.float32(-1e30))
            mloc = jnp.max(s, axis=1, keepdims=True)
            mnew = jnp.maximum(m, mloc)
            alpha = jnp.exp(m - mnew)
            p = jnp.exp(s - mnew)
            lnew = l * alpha + jnp.sum(p, axis=1, keepdims=True)
            accnew = acc * alpha + jnp.dot(p.astype(jnp.bfloat16), vs,
                                           preferred_element_type=jnp.float32)
            return mnew, lnew, accnew

        m0 = jnp.full((BT, 1), -1e30, jnp.float32)
        l0 = jnp.zeros((BT, 1), jnp.float32)
        a0 = jnp.zeros((BT, HD), jnp.float32)
        m, l, acc = lax.fori_loop(0, nb, body, (m0, l0, a0))
        o_ref[hh] = acc / l


def _attention(q3d, k3d, v3d):
    return pl.pallas_call(
        _attn_kernel,
        grid=(NT,),
        in_specs=[
            pl.BlockSpec((H, BT, HD), lambda i: (0, i, 0)),
            pl.BlockSpec((KV, T, HD), lambda i: (0, 0, 0)),
            pl.BlockSpec((KV, T, HD), lambda i: (0, 0, 0)),
        ],
        out_specs=pl.BlockSpec((H, BT, HD), lambda i: (0, i, 0)),
        out_shape=jax.ShapeDtypeStruct((H, T, HD), jnp.float32),
    )(q3d, k3d, v3d)


def _post_attn_kernel(o_ref, res_ref, wo_ref, ln2_ref, gw_ref,
                      x1_ref, h2_ref, topw_ref, topi_ref, rank_ref, cnt_ref,
                      carry):
    i = pl.program_id(0)

    @pl.when(i == 0)
    def _():
        carry[...] = jnp.zeros((1, E), jnp.float32)

    o2d = jnp.concatenate([o_ref[hh] for hh in range(H)], axis=1)
    x1 = res_ref[...] + jnp.dot(o2d.astype(jnp.bfloat16), wo_ref[...],
                                preferred_element_type=jnp.float32)
    h2 = _rms(x1, ln2_ref[...])
    x1_ref[...] = x1
    h2_ref[...] = h2
    lg = jnp.dot(h2, gw_ref[...], preferred_element_type=jnp.float32)

    m = jnp.max(lg, axis=1, keepdims=True)
    p = jnp.exp(lg - m)
    p = p / jnp.sum(p, axis=1, keepdims=True)
    ii = lax.broadcasted_iota(jnp.int32, (BT, E), 1)
    m1 = jnp.max(p, axis=1, keepdims=True)
    i1 = jnp.min(jnp.where(p == m1, ii, E), axis=1, keepdims=True)
    p2 = jnp.where(ii == i1, -1.0, p)
    m2 = jnp.max(p2, axis=1, keepdims=True)
    i2 = jnp.min(jnp.where(p2 == m2, ii, E), axis=1, keepdims=True)
    s = m1 + m2
    topw_ref[:, 0:1] = m1 / s
    topw_ref[:, 1:2] = m2 / s
    topi_ref[:, 0:1] = i1
    topi_ref[:, 1:2] = i2

    oh1 = (ii == i1).astype(jnp.float32)
    oh2 = (ii == i2).astype(jnp.float32)
    ind = oh1 + oh2
    rr = lax.broadcasted_iota(jnp.int32, (BT, BT), 0)
    cc = lax.broadcasted_iota(jnp.int32, (BT, BT), 1)
    lstrict = (rr > cc).astype(jnp.float32)
    rex = jnp.dot(lstrict, ind, preferred_element_type=jnp.float32) + carry[...]
    rank_ref[:, 0:1] = jnp.sum(rex * oh1, axis=1, keepdims=True).astype(jnp.int32)
    rank_ref[:, 1:2] = jnp.sum(rex * oh2, axis=1, keepdims=True).astype(jnp.int32)
    newc = carry[...] + jnp.sum(ind, axis=0, keepdims=True)
    carry[...] = newc
    cnt_ref[...] = newc


def _post_attn(o3d, res, w_o, ln2_w, gate_w):
    return pl.pallas_call(
        _post_attn_kernel,
        grid=(NT,),
        in_specs=[
            pl.BlockSpec((H, BT, HD), lambda i: (0, i, 0)),
            pl.BlockSpec((BT, D), lambda i: (i, 0)),
            pl.BlockSpec((H * HD, D), lambda i: (0, 0)),
            pl.BlockSpec((1, D), lambda i: (0, 0)),
            pl.BlockSpec((D, E), lambda i: (0, 0)),
        ],
        out_specs=[
            pl.BlockSpec((BT, D), lambda i: (i, 0)),
            pl.BlockSpec((BT, D), lambda i: (i, 0)),
            pl.BlockSpec((BT, K), lambda i: (i, 0)),
            pl.BlockSpec((BT, K), lambda i: (i, 0)),
            pl.BlockSpec((BT, K), lambda i: (i, 0)),
            pl.BlockSpec((1, E), lambda i: (0, 0)),
        ],
        out_shape=[
            jax.ShapeDtypeStruct((T, D), jnp.float32),
            jax.ShapeDtypeStruct((T, D), jnp.float32),
            jax.ShapeDtypeStruct((T, K), jnp.float32),
            jax.ShapeDtypeStruct((T, K), jnp.int32),
            jax.ShapeDtypeStruct((T, K), jnp.int32),
            jax.ShapeDtypeStruct((1, E), jnp.float32),
        ],
        scratch_shapes=[pltpu.VMEM((1, E), jnp.float32)],
    )(o3d, res, w_o.astype(jnp.bfloat16), ln2_w.reshape(1, D), gate_w)


BLK = 256
NPAD = T * K + E * BLK
NBLK = NPAD // BLK


def _meta_kernel(cnt_ref, topi_ref, rank_ref, pos_ref, bexp_ref, bval_ref):
    ci = cnt_ref[...].astype(jnp.int32)
    padded = ((ci + (BLK - 1)) // BLK) * BLK
    ie = lax.broadcasted_iota(jnp.int32, (E, E), 0)
    je = lax.broadcasted_iota(jnp.int32, (E, E), 1)
    ustrict = (ie < je).astype(jnp.float32)
    off = jnp.dot(padded.astype(jnp.float32), ustrict,
                  preferred_element_type=jnp.float32).astype(jnp.int32)
    ends = off + padded
    total = jnp.sum(padded)

    topi = topi_ref[...]
    offsel = jnp.zeros((T, K), jnp.int32)
    for e in range(E):
        offsel = offsel + jnp.where(topi == e, off[0, e], 0)
    pos_ref[...] = rank_ref[...] + offsel

    bi = lax.broadcasted_iota(jnp.int32, (1, NBLK), 1) * BLK
    be = jnp.zeros((1, NBLK), jnp.int32)
    for e in range(E):
        be = be + jnp.where(bi >= ends[0, e], 1, 0)
    bexp_ref[...] = jnp.minimum(be, E - 1)
    bval_ref[...] = jnp.where(bi < total, 1, 0)


def _meta(cnt, topi, rank):
    return pl.pallas_call(
        _meta_kernel,
        grid=(1,),
        in_specs=[
            pl.BlockSpec((1, E), lambda i: (0, 0)),
            pl.BlockSpec((T, K), lambda i: (0, 0)),
            pl.BlockSpec((T, K), lambda i: (0, 0)),
        ],
        out_specs=[
            pl.BlockSpec((T, K), lambda i: (0, 0)),
            pl.BlockSpec((1, NBLK), lambda i: (0, 0)),
            pl.BlockSpec((1, NBLK), lambda i: (0, 0)),
        ],
        out_shape=[
            jax.ShapeDtypeStruct((T, K), jnp.int32),
            jax.ShapeDtypeStruct((1, NBLK), jnp.int32),
            jax.ShapeDtypeStruct((1, NBLK), jnp.int32),
        ],
    )(cnt, topi, rank)


_NC = 2
_NS = 16
_NW = _NC * _NS
TPW = T // _NW
@functools.cache
def _sc_mesh():
    return plsc.VectorSubcoreMesh(core_axis_name="c", subcore_axis_name="s",
                                  num_cores=_NC, num_subcores=_NS)


def _sc_wid():
    return lax.axis_index("s") * _NC + lax.axis_index("c")


def _dispatch_scatter(h2, pos0, pos1):
    @functools.partial(
        pl.kernel,
        out_type=jax.ShapeDtypeStruct((NPAD, D), jnp.float32),
        mesh=_sc_mesh(),
        scratch_types=[
            pltpu.VMEM((TPW,), jnp.int32),
            pltpu.VMEM((TPW,), jnp.int32),
            pltpu.VMEM((TPW, D), jnp.float32),
            pltpu.SemaphoreType.DMA,
        ],
    )
    def run(h2_hbm, p0_hbm, p1_hbm, xg_hbm, idx0_v, idx1_v, rows_v, sem):
        base = _sc_wid() * TPW
        pltpu.sync_copy(p0_hbm.at[pl.ds(base, TPW)], idx0_v)
        pltpu.sync_copy(p1_hbm.at[pl.ds(base, TPW)], idx1_v)
        pltpu.sync_copy(h2_hbm.at[pl.ds(base, TPW)], rows_v)
        pltpu.async_copy(rows_v, xg_hbm.at[idx0_v], sem).wait()
        pltpu.async_copy(rows_v, xg_hbm.at[idx1_v], sem).wait()

    return run(h2, pos0, pos1)


def _combine_gather(yrows, pos0, pos1):
    @functools.partial(
        pl.kernel,
        out_type=jax.ShapeDtypeStruct((K, T, D), jnp.float32),
        mesh=_sc_mesh(),
        scratch_types=[
            pltpu.VMEM((TPW,), jnp.int32),
            pltpu.VMEM((TPW, D), jnp.float32),
            pltpu.SemaphoreType.DMA,
        ],
    )
    def run(y_hbm, p0_hbm, p1_hbm, yg_hbm, idx_v, rows_v, sem):
        base = _sc_wid() * TPW
        pltpu.sync_copy(p0_hbm.at[pl.ds(base, TPW)], idx_v)
        pltpu.async_copy(y_hbm.at[idx_v], rows_v, sem).wait()
        pltpu.sync_copy(rows_v, yg_hbm.at[0, pl.ds(base, TPW)])
        pltpu.sync_copy(p1_hbm.at[pl.ds(base, TPW)], idx_v)
        pltpu.async_copy(y_hbm.at[idx_v], rows_v, sem).wait()
        pltpu.sync_copy(rows_v, yg_hbm.at[1, pl.ds(base, TPW)])

    return run(yrows, pos0, pos1)


def _gemm_kernel(bexp_ref, bval_ref, xg_ref, wgu_ref, wd_ref, y_ref):
    b = pl.program_id(0)

    @pl.when(bval_ref[b] == 1)
    def _():
        xb = xg_ref[...]
        gu = jnp.dot(xb, wgu_ref[0], preferred_element_type=jnp.float32)
        g = gu[:, :FF]
        u = gu[:, FF:]
        act = g * (1.0 / (1.0 + jnp.exp(-g))) * u
        y_ref[...] = jnp.dot(act, wd_ref[0], preferred_element_type=jnp.float32)


def _grouped_gemm(bexp, bval, xg, w_gate_up, w_down):
    grid_spec = pltpu.PrefetchScalarGridSpec(
        num_scalar_prefetch=2,
        grid=(NBLK,),
        in_specs=[
            pl.BlockSpec((BLK, D), lambda b, be, bv: (b, 0)),
            pl.BlockSpec((1, D, 2 * FF), lambda b, be, bv: (be[b], 0, 0)),
            pl.BlockSpec((1, FF, D), lambda b, be, bv: (be[b], 0, 0)),
        ],
        out_specs=pl.BlockSpec((BLK, D), lambda b, be, bv: (b, 0)),
    )
    return pl.pallas_call(
        _gemm_kernel,
        grid_spec=grid_spec,
        out_shape=jax.ShapeDtypeStruct((NPAD, D), jnp.float32),
    )(bexp, bval, xg, w_gate_up, w_down)


def _combine_kernel(x1_ref, yg0_ref, yg1_ref, topw_ref, out_ref):
    w0 = topw_ref[:, 0:1]
    w1 = topw_ref[:, 1:2]
    out_ref[...] = x1_ref[...] + w0 * yg0_ref[0] + w1 * yg1_ref[0]


def _combine(x1, yg, topw):
    return pl.pallas_call(
        _combine_kernel,
        grid=(NT,),
        in_specs=[
            pl.BlockSpec((BT, D), lambda i: (i, 0)),
            pl.BlockSpec((1, BT, D), lambda i: (0, i, 0)),
            pl.BlockSpec((1, BT, D), lambda i: (1, i, 0)),
            pl.BlockSpec((BT, K), lambda i: (i, 0)),
        ],
        out_specs=pl.BlockSpec((BT, D), lambda i: (i, 0)),
        out_shape=jax.ShapeDtypeStruct((T, D), jnp.float32),
    )(x1, yg, yg, topw)


@jax.jit
def _layer(positions, hidden_states, w_qkv, w_o, q_norm_w, k_norm_w,
           ln1_w, ln2_w, gate_w, w_gate_up, w_down):
    q3d, k3d, v3d = _pre_attn(hidden_states, w_qkv, ln1_w, q_norm_w,
                              k_norm_w, positions)
    o3d = _attention(q3d, k3d, v3d)
    x1, h2, topw, topi, rank, cnt = _post_attn(o3d, hidden_states, w_o,
                                               ln2_w, gate_w)
    pos, bexp, bval = _meta(cnt, topi, rank)
    post = pos.T
    pos0 = post[0]
    pos1 = post[1]
    xg = _dispatch_scatter(h2, pos0, pos1)
    yrows = _grouped_gemm(bexp.reshape(NBLK), bval.reshape(NBLK),
                          xg, w_gate_up, w_down)
    yg = _combine_gather(yrows, pos0, pos1)
    return _combine(x1, yg, topw)


def kernel(positions, hidden_states, w_qkv, w_o, q_norm_w, k_norm_w,
           ln1_w, ln2_w, gate_w, w_gate_up, w_down):
    return _layer(positions, hidden_states, w_qkv, w_o, q_norm_w, k_norm_w,
                  ln1_w, ln2_w, gate_w, w_gate_up, w_down)


# speedup vs baseline: 1.0044x; 1.0044x over previous
import functools
import jax
import jax.numpy as jnp
from jax import lax
from jax.experimental import pallas as pl
from jax.experimental.pallas import tpu as pltpu
from jax.experimental.pallas import tpu_sc as plsc

T = 2048
D = 1024
H = 16
KV = 4
HD = 64
E = 8
K = 2
FF = 1024
EPS = 1e-6
THETA = 10000.0

BT = 256
NT = T // BT
HALF = HD // 2


def _rms(x, w, eps=EPS):
    var = jnp.mean(x * x, axis=-1, keepdims=True)
    return x * lax.rsqrt(var + eps) * w


def _rope_tables(pos, width):
    ci = lax.broadcasted_iota(jnp.int32, (1, width), 1)
    fidx = (ci % HALF).astype(jnp.float32)
    inv_freq = jnp.exp(fidx * (-2.0 * jnp.log(THETA) / HD))
    freqs = pos * inv_freq
    return jnp.cos(freqs), jnp.sin(freqs)


def _rot_matrix(width):
    rm = lax.broadcasted_iota(jnp.int32, (width, width), 0)
    cm = lax.broadcasted_iota(jnp.int32, (width, width), 1)
    lo = ((cm % HD) < HALF) & (rm == cm + HALF)
    hi = ((cm % HD) >= HALF) & (rm == cm - HALF)
    return jnp.where(lo, -1.0, 0.0) + jnp.where(hi, 1.0, 0.0)


def _headwise_rms_rope(t, nheads, wtiled, pos):
    width = nheads * HD
    bd = (lax.broadcasted_iota(jnp.int32, (width, nheads), 0) // HD ==
          lax.broadcasted_iota(jnp.int32, (width, nheads), 1)).astype(jnp.float32)
    var = jnp.dot(t * t, bd, preferred_element_type=jnp.float32) * (1.0 / HD)
    scale = jnp.dot(lax.rsqrt(var + EPS), bd.T, preferred_element_type=jnp.float32)
    th = t * scale * wtiled
    rot = jnp.dot(th, _rot_matrix(width), preferred_element_type=jnp.float32)
    cosf, sinf = _rope_tables(pos, width)
    return th * cosf + rot * sinf


def _pre_attn_kernel(x_ref, wqkv_ref, ln1_ref, qn_ref, kn_ref, pos_ref,
                     q_ref, k_ref, v_ref):
    x = x_ref[...]
    h = _rms(x, ln1_ref[...])
    qkv = jnp.dot(h.astype(jnp.bfloat16), wqkv_ref[...],
                  preferred_element_type=jnp.float32)
    pos = pos_ref[...].astype(jnp.float32)

    qr = _headwise_rms_rope(qkv[:, :H * HD], H, qn_ref[...], pos)
    kr = _headwise_rms_rope(qkv[:, H * HD:(H + KV) * HD], KV, kn_ref[...], pos)
    for hh in range(H):
        q_ref[hh, :, :] = qr[:, hh * HD:(hh + 1) * HD].astype(jnp.bfloat16)
    for hh in range(KV):
        k_ref[hh, :, :] = kr[:, hh * HD:(hh + 1) * HD].astype(jnp.bfloat16)
        v_ref[hh, :, :] = qkv[:, (H + KV) * HD + hh * HD:
                              (H + KV) * HD + (hh + 1) * HD].astype(jnp.bfloat16)


def _pre_attn(x, w_qkv, ln1_w, q_norm_w, k_norm_w, positions):
    return pl.pallas_call(
        _pre_attn_kernel,
        grid=(NT,),
        in_specs=[
            pl.BlockSpec((BT, D), lambda i: (i, 0)),
            pl.BlockSpec((D, (H + 2 * KV) * HD), lambda i: (0, 0)),
            pl.BlockSpec((1, D), lambda i: (0, 0)),
            pl.BlockSpec((1, H * HD), lambda i: (0, 0)),
            pl.BlockSpec((1, KV * HD), lambda i: (0, 0)),
            pl.BlockSpec((BT, 1), lambda i: (i, 0)),
        ],
        out_specs=[
            pl.BlockSpec((H, BT, HD), lambda i: (0, i, 0)),
            pl.BlockSpec((KV, BT, HD), lambda i: (0, i, 0)),
            pl.BlockSpec((KV, BT, HD), lambda i: (0, i, 0)),
        ],
        out_shape=[
            jax.ShapeDtypeStruct((H, T, HD), jnp.bfloat16),
            jax.ShapeDtypeStruct((KV, T, HD), jnp.bfloat16),
            jax.ShapeDtypeStruct((KV, T, HD), jnp.bfloat16),
        ],
    )(x, w_qkv.astype(jnp.bfloat16), ln1_w.reshape(1, D),
      jnp.tile(q_norm_w, H).reshape(1, H * HD),
      jnp.tile(k_norm_w, KV).reshape(1, KV * HD), positions.reshape(T, 1))


BKV = 1024


def _attn_kernel(q_ref, k_ref, v_ref, o_ref):
    iq = pl.program_id(0)
    nb = (iq * BT + BT + BKV - 1) // BKV

    for hh in range(H):
        kvh = hh // (H // KV)
        q = q_ref[hh]

        def body(j, carry):
            m, l, acc = carry
            ks = k_ref[kvh, pl.ds(j * BKV, BKV), :]
            vs = v_ref[kvh, pl.ds(j * BKV, BKV), :]
            s = lax.dot_general(q, ks, (((1,), (1,)), ((), ())),
                                preferred_element_type=jnp.float32)
            s = s * (HD ** -0.5)
            row = iq * BT + lax.broadcasted_iota(jnp.int32, (BT, BKV), 0)
            col = j * BKV + lax.broadcasted_iota(jnp.int32, (BT, BKV), 1)
            s = jnp.where(row >= col, s, jnp.float32(-1e30))
            mloc = jnp.max(s, axis=1, keepdims=True)
            mnew = jnp.maximum(m, mloc)
            alpha = jnp.exp(m - mnew)
            p = jnp.exp(s - mnew)
            lnew = l * alpha + jnp.sum(p, axis=1, keepdims=True)
            accnew = acc * alpha + jnp.dot(p.astype(jnp.bfloat16), vs,
                                           preferred_element_type=jnp.float32)
            return mnew, lnew, accnew

        m0 = jnp.full((BT, 1), -1e30, jnp.float32)
        l0 = jnp.zeros((BT, 1), jnp.float32)
        a0 = jnp.zeros((BT, HD), jnp.float32)
        m, l, acc = lax.fori_loop(0, nb, body, (m0, l0, a0))
        o_ref[hh] = acc / l


def _attention(q3d, k3d, v3d):
    return pl.pallas_call(
        _attn_kernel,
        grid=(NT,),
        in_specs=[
            pl.BlockSpec((H, BT, HD), lambda i: (0, i, 0)),
            pl.BlockSpec((KV, T, HD), lambda i: (0, 0, 0)),
            pl.BlockSpec((KV, T, HD), lambda i: (0, 0, 0)),
        ],
        out_specs=pl.BlockSpec((H, BT, HD), lambda i: (0, i, 0)),
        out_shape=jax.ShapeDtypeStruct((H, T, HD), jnp.float32),
    )(q3d, k3d, v3d)


def _post_attn_kernel(o_ref, res_ref, wo_ref, ln2_ref, gw_ref,
                      x1_ref, h2_ref, topw_ref, topi_ref, rank_ref, cnt_ref,
                      carry):
    i = pl.program_id(0)

    @pl.when(i == 0)
    def _():
        carry[...] = jnp.zeros((1, E), jnp.float32)

    o2d = jnp.concatenate([o_ref[hh] for hh in range(H)], axis=1)
    x1 = res_ref[...] + jnp.dot(o2d.astype(jnp.bfloat16), wo_ref[...],
                                preferred_element_type=jnp.float32)
    h2 = _rms(x1, ln2_ref[...])
    x1_ref[...] = x1
    h2_ref[...] = h2
    lg = jnp.dot(h2, gw_ref[...], preferred_element_type=jnp.float32)

    m = jnp.max(lg, axis=1, keepdims=True)
    p = jnp.exp(lg - m)
    p = p / jnp.sum(p, axis=1, keepdims=True)
    ii = lax.broadcasted_iota(jnp.int32, (BT, E), 1)
    m1 = jnp.max(p, axis=1, keepdims=True)
    i1 = jnp.min(jnp.where(p == m1, ii, E), axis=1, keepdims=True)
    p2 = jnp.where(ii == i1, -1.0, p)
    m2 = jnp.max(p2, axis=1, keepdims=True)
    i2 = jnp.min(jnp.where(p2 == m2, ii, E), axis=1, keepdims=True)
    s = m1 + m2
    topw_ref[:, 0:1] = m1 / s
    topw_ref[:, 1:2] = m2 / s
    topi_ref[:, 0:1] = i1
    topi_ref[:, 1:2] = i2

    oh1 = (ii == i1).astype(jnp.float32)
    oh2 = (ii == i2).astype(jnp.float32)
    ind = oh1 + oh2
    rr = lax.broadcasted_iota(jnp.int32, (BT, BT), 0)
    cc = lax.broadcasted_iota(jnp.int32, (BT, BT), 1)
    lstrict = (rr > cc).astype(jnp.float32)
    rex = jnp.dot(lstrict, ind, preferred_element_type=jnp.float32) + carry[...]
    rank_ref[:, 0:1] = jnp.sum(rex * oh1, axis=1, keepdims=True).astype(jnp.int32)
    rank_ref[:, 1:2] = jnp.sum(rex * oh2, axis=1, keepdims=True).astype(jnp.int32)
    newc = carry[...] + jnp.sum(ind, axis=0, keepdims=True)
    carry[...] = newc
    cnt_ref[...] = newc


def _post_attn(o3d, res, w_o, ln2_w, gate_w):
    return pl.pallas_call(
        _post_attn_kernel,
        grid=(NT,),
        in_specs=[
            pl.BlockSpec((H, BT, HD), lambda i: (0, i, 0)),
            pl.BlockSpec((BT, D), lambda i: (i, 0)),
            pl.BlockSpec((H * HD, D), lambda i: (0, 0)),
            pl.BlockSpec((1, D), lambda i: (0, 0)),
            pl.BlockSpec((D, E), lambda i: (0, 0)),
        ],
        out_specs=[
            pl.BlockSpec((BT, D), lambda i: (i, 0)),
            pl.BlockSpec((BT, D), lambda i: (i, 0)),
            pl.BlockSpec((BT, K), lambda i: (i, 0)),
            pl.BlockSpec((BT, K), lambda i: (i, 0)),
            pl.BlockSpec((BT, K), lambda i: (i, 0)),
            pl.BlockSpec((1, E), lambda i: (0, 0)),
        ],
        out_shape=[
            jax.ShapeDtypeStruct((T, D), jnp.float32),
            jax.ShapeDtypeStruct((T, D), jnp.float32),
            jax.ShapeDtypeStruct((T, K), jnp.float32),
            jax.ShapeDtypeStruct((T, K), jnp.int32),
            jax.ShapeDtypeStruct((T, K), jnp.int32),
            jax.ShapeDtypeStruct((1, E), jnp.float32),
        ],
        scratch_shapes=[pltpu.VMEM((1, E), jnp.float32)],
    )(o3d, res, w_o.astype(jnp.bfloat16), ln2_w.reshape(1, D), gate_w)


BLK = 256
NPAD = T * K + E * BLK
NBLK = NPAD // BLK


def _meta_kernel(cnt_ref, topi_ref, rank_ref, pos_ref, bexp_ref, bval_ref):
    ci = cnt_ref[...].astype(jnp.int32)
    padded = ((ci + (BLK - 1)) // BLK) * BLK
    ie = lax.broadcasted_iota(jnp.int32, (E, E), 0)
    je = lax.broadcasted_iota(jnp.int32, (E, E), 1)
    ustrict = (ie < je).astype(jnp.float32)
    off = jnp.dot(padded.astype(jnp.float32), ustrict,
                  preferred_element_type=jnp.float32).astype(jnp.int32)
    ends = off + padded
    total = jnp.sum(padded)

    topi = topi_ref[...]
    offsel = jnp.zeros((T, K), jnp.int32)
    for e in range(E):
        offsel = offsel + jnp.where(topi == e, off[0, e], 0)
    pos_ref[...] = rank_ref[...] + offsel

    bi = lax.broadcasted_iota(jnp.int32, (1, NBLK), 1) * BLK
    be = jnp.zeros((1, NBLK), jnp.int32)
    for e in range(E):
        be = be + jnp.where(bi >= ends[0, e], 1, 0)
    bexp_ref[...] = jnp.minimum(be, E - 1)
    bval_ref[...] = jnp.where(bi < total, 1, 0)


def _meta(cnt, topi, rank):
    return pl.pallas_call(
        _meta_kernel,
        grid=(1,),
        in_specs=[
            pl.BlockSpec((1, E), lambda i: (0, 0)),
            pl.BlockSpec((T, K), lambda i: (0, 0)),
            pl.BlockSpec((T, K), lambda i: (0, 0)),
        ],
        out_specs=[
            pl.BlockSpec((T, K), lambda i: (0, 0)),
            pl.BlockSpec((1, NBLK), lambda i: (0, 0)),
            pl.BlockSpec((1, NBLK), lambda i: (0, 0)),
        ],
        out_shape=[
            jax.ShapeDtypeStruct((T, K), jnp.int32),
            jax.ShapeDtypeStruct((1, NBLK), jnp.int32),
            jax.ShapeDtypeStruct((1, NBLK), jnp.int32),
        ],
    )(cnt, topi, rank)


_NC = 2
_NS = 16
_NW = _NC * _NS
TPW = T // _NW
@functools.cache
def _sc_mesh():
    return plsc.VectorSubcoreMesh(core_axis_name="c", subcore_axis_name="s",
                                  num_cores=_NC, num_subcores=_NS)


def _sc_wid():
    return lax.axis_index("s") * _NC + lax.axis_index("c")


def _dispatch_scatter(h2, pos0, pos1):
    @functools.partial(
        pl.kernel,
        out_type=jax.ShapeDtypeStruct((NPAD, D), jnp.float32),
        mesh=_sc_mesh(),
        scratch_types=[
            pltpu.VMEM((TPW,), jnp.int32),
            pltpu.VMEM((TPW,), jnp.int32),
            pltpu.VMEM((TPW, D), jnp.float32),
            pltpu.SemaphoreType.DMA,
        ],
    )
    def run(h2_hbm, p0_hbm, p1_hbm, xg_hbm, idx0_v, idx1_v, rows_v, sem):
        base = _sc_wid() * TPW
        pltpu.sync_copy(p0_hbm.at[pl.ds(base, TPW)], idx0_v)
        pltpu.sync_copy(p1_hbm.at[pl.ds(base, TPW)], idx1_v)
        pltpu.sync_copy(h2_hbm.at[pl.ds(base, TPW)], rows_v)
        pltpu.async_copy(rows_v, xg_hbm.at[idx0_v], sem).wait()
        pltpu.async_copy(rows_v, xg_hbm.at[idx1_v], sem).wait()

    return run(h2, pos0, pos1)


def _combine_gather(yrows, pos0, pos1):
    @functools.partial(
        pl.kernel,
        out_type=jax.ShapeDtypeStruct((K, T, D), jnp.float32),
        mesh=_sc_mesh(),
        scratch_types=[
            pltpu.VMEM((TPW,), jnp.int32),
            pltpu.VMEM((TPW, D), jnp.float32),
            pltpu.SemaphoreType.DMA,
        ],
    )
    def run(y_hbm, p0_hbm, p1_hbm, yg_hbm, idx_v, rows_v, sem):
        base = _sc_wid() * TPW
        pltpu.sync_copy(p0_hbm.at[pl.ds(base, TPW)], idx_v)
        pltpu.async_copy(y_hbm.at[idx_v], rows_v, sem).wait()
        pltpu.sync_copy(rows_v, yg_hbm.at[0, pl.ds(base, TPW)])
        pltpu.sync_copy(p1_hbm.at[pl.ds(base, TPW)], idx_v)
        pltpu.async_copy(y_hbm.at[idx_v], rows_v, sem).wait()
        pltpu.sync_copy(rows_v, yg_hbm.at[1, pl.ds(base, TPW)])

    return run(yrows, pos0, pos1)


def _gemm_kernel(bexp_ref, bval_ref, xg_ref, wgu_ref, wd_ref, y_ref):
    b = pl.program_id(0)

    @pl.when(bval_ref[b] == 1)
    def _():
        xb = xg_ref[...]
        gu = jnp.dot(xb, wgu_ref[0], preferred_element_type=jnp.float32)
        g = gu[:, :FF]
        u = gu[:, FF:]
        act = g * (1.0 / (1.0 + jnp.exp(-g))) * u
        y_ref[...] = jnp.dot(act, wd_ref[0], preferred_element_type=jnp.float32)


def _grouped_gemm(bexp, bval, xg, w_gate_up, w_down):
    grid_spec = pltpu.PrefetchScalarGridSpec(
        num_scalar_prefetch=2,
        grid=(NBLK,),
        in_specs=[
            pl.BlockSpec((BLK, D), lambda b, be, bv: (b, 0)),
            pl.BlockSpec((1, D, 2 * FF), lambda b, be, bv: (be[b], 0, 0)),
            pl.BlockSpec((1, FF, D), lambda b, be, bv: (be[b], 0, 0)),
        ],
        out_specs=pl.BlockSpec((BLK, D), lambda b, be, bv: (b, 0)),
    )
    return pl.pallas_call(
        _gemm_kernel,
        grid_spec=grid_spec,
        out_shape=jax.ShapeDtypeStruct((NPAD, D), jnp.float32),
    )(bexp, bval, xg, w_gate_up, w_down)


def _combine_kernel(x1_ref, yg0_ref, yg1_ref, topw_ref, out_ref):
    w0 = topw_ref[:, 0:1]
    w1 = topw_ref[:, 1:2]
    out_ref[...] = x1_ref[...] + w0 * yg0_ref[0] + w1 * yg1_ref[0]


def _combine(x1, yg, topw):
    return pl.pallas_call(
        _combine_kernel,
        grid=(NT,),
        in_specs=[
            pl.BlockSpec((BT, D), lambda i: (i, 0)),
            pl.BlockSpec((1, BT, D), lambda i: (0, i, 0)),
            pl.BlockSpec((1, BT, D), lambda i: (1, i, 0)),
            pl.BlockSpec((BT, K), lambda i: (i, 0)),
        ],
        out_specs=pl.BlockSpec((BT, D), lambda i: (i, 0)),
        out_shape=jax.ShapeDtypeStruct((T, D), jnp.float32),
    )(x1, yg, yg, topw)


@jax.jit
def _layer(positions, hidden_states, w_qkv, w_o, q_norm_w, k_norm_w,
           ln1_w, ln2_w, gate_w, w_gate_up, w_down):
    q3d, k3d, v3d = _pre_attn(hidden_states, w_qkv, ln1_w, q_norm_w,
                              k_norm_w, positions)
    o3d = _attention(q3d, k3d, v3d)
    x1, h2, topw, topi, rank, cnt = _post_attn(o3d, hidden_states, w_o,
                                               ln2_w, gate_w)
    pos, bexp, bval = _meta(cnt, topi, rank)
    post = pos.T
    pos0 = post[0]
    pos1 = post[1]
    xg = _dispatch_scatter(h2, pos0, pos1)
    yrows = _grouped_gemm(bexp.reshape(NBLK), bval.reshape(NBLK),
                          xg, w_gate_up, w_down)
    yg = _combine_gather(yrows, pos0, pos1)
    return _combine(x1, yg, topw)


def kernel(positions, hidden_states, w_qkv, w_o, q_norm_w, k_norm_w,
           ln1_w, ln2_w, gate_w, w_gate_up, w_down):
    return _layer(positions, hidden_states, w_qkv, w_o, q_norm_w, k_norm_w,
                  ln1_w, ln2_w, gate_w, w_gate_up, w_down)
